# baseline (speedup 1.0000x reference)
import jax
import jax.numpy as jnp
from jax import lax
from jax.experimental import pallas as pl
from jax.experimental.pallas import tpu as pltpu

S, B, D, H, A = 2048, 16, 64, 128, 16
NC = 4
CL = S // NC
B4 = NC * B
BF = (NC - 1) * B
T_BLK = 64
N_BLK = CL // T_BLK
FC = 64


def _gru_kernel(x_ref, draw_ref, ks_ref, ksn_ref, xF_ref, drawF_ref,
                h0_ref, Wi_ref, Wh_ref, bfold_ref, bhn_ref, Wout_ref,
                bout_ref, out_ref, h_ref, gi_ref, hs_ref, hfix_ref,
                gx_ref):
    i = pl.program_id(0)

    @pl.when(i == 0)
    def _():
        d0 = draw_ref[0, :B].astype(jnp.float32)[:, None]
        hA = h0_ref[0] * (1.0 - d0)
        h_ref[...] = jnp.concatenate(
            [hA, jnp.zeros((B4 - B, H), jnp.float32)], axis=0)

    x = x_ref[...].reshape(T_BLK * B4, D)
    gi = jnp.dot(x, Wi_ref[...], preferred_element_type=jnp.float32)
    gi_ref[...] = (gi + bfold_ref[0]).reshape(T_BLK, B4, 3 * H)

    Wh = Wh_ref[...]
    Wh16 = Wh.astype(jnp.bfloat16)
    bhn = bhn_ref[0]

    def step(t, h):
        m = jnp.dot(h.astype(jnp.bfloat16), Wh16,
                    preferred_element_type=jnp.float32)
        k = 1.0 - draw_ref[i * T_BLK + t, :].astype(jnp.float32)[:, None]
        gh = m * k
        h_m = h * k
        gi_t = gi_ref[t]
        r = jax.nn.sigmoid(gi_t[:, :H] + gh[:, :H])
        z = jax.nn.sigmoid(gi_t[:, H:2 * H] + gh[:, H:2 * H])
        n = jnp.tanh(gi_t[:, 2 * H:] + r * (gh[:, 2 * H:] + bhn))
        h_new = n + z * (h_m - n)
        hs_ref[t] = h_new
        return h_new

    h_final = lax.fori_loop(0, T_BLK, step, h_ref[...], unroll=32)
    h_ref[...] = h_final

    hs = hs_ref[...].reshape(T_BLK * B4, H)
    o = jnp.dot(hs, Wout_ref[...], preferred_element_type=jnp.float32)
    o = (o + bout_ref[0]).reshape(T_BLK, B4, A)
    out_ref[pl.ds(i * T_BLK, T_BLK)] = o

    @pl.when(i == N_BLK - 1)
    def _():
        dF = drawF_ref[...]
        tv = lax.broadcasted_iota(jnp.int32, (CL, BF), 0)
        firstF = jnp.min(jnp.where(dF > 0, tv, CL), axis=0)
        lidx = lax.broadcasted_iota(jnp.int32, (BF,), 0)
        L1 = jnp.max(jnp.where(lidx < B, firstF, 0))
        L2 = jnp.max(jnp.where((lidx >= B) & (lidx < 2 * B), firstF, 0))
        L3 = jnp.max(jnp.where(lidx >= 2 * B, firstF, 0))
        L = jnp.max(firstF)
        flag1 = L1 == CL
        flag2 = L2 == CL

        Wi = Wi_ref[...]
        bfold = bfold_ref[0]
        Wout = Wout_ref[...]
        bout = bout_ref[0]

        def gate_step(gx, gh, h_m):
            r = jax.nn.sigmoid(gx[:, :H] + gh[:, :H])
            z = jax.nn.sigmoid(gx[:, H:2 * H] + gh[:, H:2 * H])
            n = jnp.tanh(gx[:, 2 * H:] + r * (gh[:, 2 * H:] + bhn))
            return n + z * (h_m - n)

        def fchunk(c, hf):
            base = c * FC
            xc = xF_ref[pl.ds(base, FC)].reshape(FC * BF, D)
            g = jnp.dot(xc, Wi, preferred_element_type=jnp.float32)
            gx_ref[...] = (g + bfold).reshape(FC, BF, 3 * H)

            def fstep(tl, hf):
                m = jnp.dot(hf, Wh, preferred_element_type=jnp.float32)
                t = base + tl
                k = 1.0 - drawF_ref[t, :].astype(jnp.float32)[:, None]
                h_new = gate_step(gx_ref[tl], m * k, hf * k)
                hfix_ref[t] = h_new
                return h_new

            def f8(j, hf):
                b8 = j * 8
                for jj in range(8):
                    hf = fstep(b8 + jj, hf)
                return hf

            n8 = jnp.minimum((L - base + 7) // 8, FC // 8)
            return lax.fori_loop(0, n8, f8, hf)

        r1_final = lax.fori_loop(0, (L + FC - 1) // FC, fchunk,
                                 h_ref[:BF, :])

        def redo(lo, start, trips):
            def rstep(t, hf):
                m = jnp.dot(hf, Wh, preferred_element_type=jnp.float32)
                gx = jnp.dot(xF_ref[t, lo:lo + B], Wi,
                             preferred_element_type=jnp.float32) + bfold
                k = 1.0 - drawF_ref[t, lo:lo + B].astype(
                    jnp.float32)[:, None]
                h_new = gate_step(gx, m * k, hf * k)
                hfix_ref[t, lo:lo + B] = h_new
                return h_new

            def r8(j, hf):
                b8 = j * 8
                for jj in range(8):
                    hf = rstep(b8 + jj, hf)
                return hf

            return lax.fori_loop(0, (trips + 7) // 8, r8, start)

        s2 = jnp.where(flag1, r1_final[:B, :], h_ref[B:2 * B, :])
        redo2_final = redo(B, s2, jnp.where(flag1, L2, 0))

        s3 = jnp.where(flag2,
                       jnp.where(flag1, redo2_final, r1_final[B:2 * B, :]),
                       h_ref[2 * B:3 * B, :])
        redo(2 * B, s3, jnp.where(flag2, L3, 0))

        of = jnp.dot(hfix_ref[...].reshape(CL * BF, H), Wout,
                     preferred_element_type=jnp.float32) + bout
        of = of.reshape(CL, BF, A)
        sel = lax.broadcasted_iota(jnp.int32, (CL, BF, A), 0) < L
        out_ref[:, B:] = jnp.where(sel, of, out_ref[:, B:])

        keep = (1 - ks_ref[...] * ksn_ref[...]).astype(jnp.float32)
        out_ref[...] = out_ref[...] * keep[:, :, None]


def _fold(a, nc):
    return jnp.swapaxes(a.reshape((nc, S // nc) + a.shape[1:]), 0, 1).reshape(
        (S // nc, nc * a.shape[1]) + a.shape[2:])


@jax.jit
def kernel(obs, hidden_states, dones, Wi, Wh, bi, bh, Wout, bout):
    x2 = obs.reshape(S, B, D)
    d2 = dones.reshape(S, B)
    split = d2.at[0, :].set(1)
    split_next = jnp.concatenate(
        [split[1:], jnp.ones((1, B), dtype=split.dtype)], axis=0)
    bfold = bi + jnp.concatenate([bh[:2 * H], jnp.zeros((H,), bh.dtype)])
    bhn = bh[2 * H:]

    xT = x2[CL:]
    dT = d2[CL:]
    foldF = lambda a: jnp.swapaxes(
        a.reshape((NC - 1, CL) + a.shape[1:]), 0, 1).reshape(
        (CL, (NC - 1) * a.shape[1]) + a.shape[2:])

    full = lambda s: pl.BlockSpec(s, lambda i: tuple(0 for _ in s))
    out = pl.pallas_call(
        _gru_kernel,
        grid=(N_BLK,),
        in_specs=[
            pl.BlockSpec((T_BLK, B4, D), lambda i: (i, 0, 0)),
            full((CL, B4)),
            full((CL, B4)),
            full((CL, B4)),
            full((CL, BF, D)),
            full((CL, BF)),
            full((1, B, H)),
            full((D, 3 * H)),
            full((H, 3 * H)),
            full((1, 3 * H)),
            full((1, H)),
            full((H, A)),
            full((1, A)),
        ],
        out_specs=pl.BlockSpec((CL, B4, A), lambda i: (0, 0, 0)),
        out_shape=jax.ShapeDtypeStruct((CL, B4, A), jnp.float32),
        scratch_shapes=[
            pltpu.VMEM((B4, H), jnp.float32),
            pltpu.VMEM((T_BLK, B4, 3 * H), jnp.float32),
            pltpu.VMEM((T_BLK, B4, H), jnp.float32),
            pltpu.VMEM((CL, BF, H), jnp.float32),
            pltpu.VMEM((FC, BF, 3 * H), jnp.float32),
        ],
    )(_fold(x2, NC), _fold(d2, NC), _fold(split, NC), _fold(split_next, NC),
      foldF(xT), foldF(dT), hidden_states,
      Wi, Wh, bfold.reshape(1, 3 * H), bhn.reshape(1, H),
      Wout, bout.reshape(1, A))
    return jnp.swapaxes(out.reshape(CL, NC, B, A), 0, 1).reshape(S * B, A)


# speedup vs baseline: 1.0192x; 1.0192x over previous
import jax
import jax.numpy as jnp
from jax import lax
from jax.experimental import pallas as pl
from jax.experimental.pallas import tpu as pltpu

S, B, D, H, A = 2048, 16, 64, 128, 16
NC = 4
CL = S // NC
B4 = NC * B
BF = (NC - 1) * B
T_BLK = 64
N_BLK = CL // T_BLK
FC = 64


def _gru_kernel(x_ref, draw_ref, ks_ref, ksn_ref, xF_ref, drawF_ref,
                h0_ref, Wi_ref, Wh_ref, bfold_ref, bhn_ref, Wout_ref,
                bout_ref, out_ref, h_ref, gi_ref, hs_ref, hfix_ref,
                gx_ref):
    i = pl.program_id(0)

    @pl.when(i == 0)
    def _():
        d0 = draw_ref[0, :B].astype(jnp.float32)[:, None]
        hA = h0_ref[0] * (1.0 - d0)
        h_ref[...] = jnp.concatenate(
            [hA, jnp.zeros((B4 - B, H), jnp.float32)], axis=0)

    x = x_ref[...].reshape(T_BLK * B4, D)
    gi = jnp.dot(x, Wi_ref[...], preferred_element_type=jnp.float32)
    gi_ref[...] = (gi + bfold_ref[0]).reshape(T_BLK, B4, 3 * H)

    Wh = Wh_ref[...]
    bhn = bhn_ref[0]

    def step(t, h):
        m = jnp.dot(h, Wh, preferred_element_type=jnp.float32)
        k = 1.0 - draw_ref[i * T_BLK + t, :].astype(jnp.float32)[:, None]
        gh = m * k
        h_m = h * k
        gi_t = gi_ref[t]
        r = jax.nn.sigmoid(gi_t[:, :H] + gh[:, :H])
        z = jax.nn.sigmoid(gi_t[:, H:2 * H] + gh[:, H:2 * H])
        n = jnp.tanh(gi_t[:, 2 * H:] + r * (gh[:, 2 * H:] + bhn))
        h_new = n + z * (h_m - n)
        hs_ref[t] = h_new
        return h_new

    h_final = lax.fori_loop(0, T_BLK, step, h_ref[...], unroll=32)
    h_ref[...] = h_final

    hs = hs_ref[...].reshape(T_BLK * B4, H)
    o = jnp.dot(hs, Wout_ref[...], preferred_element_type=jnp.float32)
    o = (o + bout_ref[0]).reshape(T_BLK, B4, A)
    out_ref[pl.ds(i * T_BLK, T_BLK)] = o

    @pl.when(i == N_BLK - 1)
    def _():
        dF = drawF_ref[...]
        tv = lax.broadcasted_iota(jnp.int32, (CL, BF), 0)
        firstF = jnp.min(jnp.where(dF > 0, tv, CL), axis=0)
        lidx = lax.broadcasted_iota(jnp.int32, (BF,), 0)
        L1 = jnp.max(jnp.where(lidx < B, firstF, 0))
        L2 = jnp.max(jnp.where((lidx >= B) & (lidx < 2 * B), firstF, 0))
        L3 = jnp.max(jnp.where(lidx >= 2 * B, firstF, 0))
        L = jnp.max(firstF)
        flag1 = L1 == CL
        flag2 = L2 == CL

        Wi = Wi_ref[...]
        bfold = bfold_ref[0]
        Wout = Wout_ref[...]
        bout = bout_ref[0]

        def gate_step(gx, gh, h_m):
            r = jax.nn.sigmoid(gx[:, :H] + gh[:, :H])
            z = jax.nn.sigmoid(gx[:, H:2 * H] + gh[:, H:2 * H])
            n = jnp.tanh(gx[:, 2 * H:] + r * (gh[:, 2 * H:] + bhn))
            return n + z * (h_m - n)

        def fchunk(c, hf):
            base = c * FC
            xc = xF_ref[pl.ds(base, FC)].reshape(FC * BF, D)
            g = jnp.dot(xc, Wi, preferred_element_type=jnp.float32)
            gx_ref[...] = (g + bfold).reshape(FC, BF, 3 * H)

            def fstep(tl, hf):
                m = jnp.dot(hf, Wh, preferred_element_type=jnp.float32)
                t = base + tl
                k = 1.0 - drawF_ref[t, :].astype(jnp.float32)[:, None]
                h_new = gate_step(gx_ref[tl], m * k, hf * k)
                hfix_ref[t] = h_new
                return h_new

            def f8(j, hf):
                b8 = j * 8
                for jj in range(8):
                    hf = fstep(b8 + jj, hf)
                return hf

            n8 = jnp.minimum((L - base + 7) // 8, FC // 8)
            return lax.fori_loop(0, n8, f8, hf)

        r1_final = lax.fori_loop(0, (L + FC - 1) // FC, fchunk,
                                 h_ref[:BF, :])

        def redo(lo, start, trips):
            def rstep(t, hf):
                m = jnp.dot(hf, Wh, preferred_element_type=jnp.float32)
                gx = jnp.dot(xF_ref[t, lo:lo + B], Wi,
                             preferred_element_type=jnp.float32) + bfold
                k = 1.0 - drawF_ref[t, lo:lo + B].astype(
                    jnp.float32)[:, None]
                h_new = gate_step(gx, m * k, hf * k)
                hfix_ref[t, lo:lo + B] = h_new
                return h_new

            def r8(j, hf):
                b8 = j * 8
                for jj in range(8):
                    hf = rstep(b8 + jj, hf)
                return hf

            return lax.fori_loop(0, (trips + 7) // 8, r8, start)

        s2 = jnp.where(flag1, r1_final[:B, :], h_ref[B:2 * B, :])
        redo2_final = redo(B, s2, jnp.where(flag1, L2, 0))

        s3 = jnp.where(flag2,
                       jnp.where(flag1, redo2_final, r1_final[B:2 * B, :]),
                       h_ref[2 * B:3 * B, :])
        redo(2 * B, s3, jnp.where(flag2, L3, 0))

        of = jnp.dot(hfix_ref[...].reshape(CL * BF, H), Wout,
                     preferred_element_type=jnp.float32) + bout
        of = of.reshape(CL, BF, A)
        sel = lax.broadcasted_iota(jnp.int32, (CL, BF, A), 0) < L
        out_ref[:, B:] = jnp.where(sel, of, out_ref[:, B:])

        keep = (1 - ks_ref[...] * ksn_ref[...]).astype(jnp.float32)
        out_ref[...] = out_ref[...] * keep[:, :, None]


def _fold(a, nc):
    return jnp.swapaxes(a.reshape((nc, S // nc) + a.shape[1:]), 0, 1).reshape(
        (S // nc, nc * a.shape[1]) + a.shape[2:])


@jax.jit
def kernel(obs, hidden_states, dones, Wi, Wh, bi, bh, Wout, bout):
    x2 = obs.reshape(S, B, D)
    d2 = dones.reshape(S, B)
    split = d2.at[0, :].set(1)
    split_next = jnp.concatenate(
        [split[1:], jnp.ones((1, B), dtype=split.dtype)], axis=0)
    bfold = bi + jnp.concatenate([bh[:2 * H], jnp.zeros((H,), bh.dtype)])
    bhn = bh[2 * H:]

    xT = x2[CL:]
    dT = d2[CL:]
    foldF = lambda a: jnp.swapaxes(
        a.reshape((NC - 1, CL) + a.shape[1:]), 0, 1).reshape(
        (CL, (NC - 1) * a.shape[1]) + a.shape[2:])

    full = lambda s: pl.BlockSpec(s, lambda i: tuple(0 for _ in s))
    out = pl.pallas_call(
        _gru_kernel,
        grid=(N_BLK,),
        in_specs=[
            pl.BlockSpec((T_BLK, B4, D), lambda i: (i, 0, 0)),
            full((CL, B4)),
            full((CL, B4)),
            full((CL, B4)),
            full((CL, BF, D)),
            full((CL, BF)),
            full((1, B, H)),
            full((D, 3 * H)),
            full((H, 3 * H)),
            full((1, 3 * H)),
            full((1, H)),
            full((H, A)),
            full((1, A)),
        ],
        out_specs=pl.BlockSpec((CL, B4, A), lambda i: (0, 0, 0)),
        out_shape=jax.ShapeDtypeStruct((CL, B4, A), jnp.float32),
        scratch_shapes=[
            pltpu.VMEM((B4, H), jnp.float32),
            pltpu.VMEM((T_BLK, B4, 3 * H), jnp.float32),
            pltpu.VMEM((T_BLK, B4, H), jnp.float32),
            pltpu.VMEM((CL, BF, H), jnp.float32),
            pltpu.VMEM((FC, BF, 3 * H), jnp.float32),
        ],
    )(_fold(x2, NC), _fold(d2, NC), _fold(split, NC), _fold(split_next, NC),
      foldF(xT), foldF(dT), hidden_states,
      Wi, Wh, bfold.reshape(1, 3 * H), bhn.reshape(1, H),
      Wout, bout.reshape(1, A))
    return jnp.swapaxes(out.reshape(CL, NC, B, A), 0, 1).reshape(S * B, A)
